# baseline (speedup 1.0000x reference)
import functools

import jax
import jax.numpy as jnp
from jax import lax
from jax.experimental import pallas as pl
from jax.experimental.pallas import tpu as pltpu
from jax.experimental.pallas import tpu_sc as plsc

NC, NS, L = 2, 16, 16
NW = NC * NS

D = 64


def _cast_chunk(rows_v, out_v, g_rows):
    iota2 = lax.iota(jnp.int32, L) * 2

    def row_body(r, _):
        ridx = jnp.full((L,), r, dtype=jnp.int32)
        for half in range(2):
            ev = plsc.load_gather(rows_v, [ridx, iota2 + half * 32])
            od = plsc.load_gather(rows_v, [ridx, iota2 + (half * 32 + 1)])
            pk = plsc.pack(ev, od, format=plsc.PackFormat.INTERLEAVED)
            out_v[pl.ds(r * D + half * 32, 32)] = pk
        return ()

    lax.fori_loop(0, g_rows, row_body, (), unroll=False)


def _body(rows_per_w, G, idx_hbm, tab_hbm, out_hbm, idx_g, rows_v, out_v, sem):
    wid = lax.axis_index("s") * NC + lax.axis_index("c")
    base = wid * rows_per_w
    num_g = rows_per_w // G

    def g_body(g, _):
        pltpu.sync_copy(idx_hbm.at[pl.ds(base + g * G, G)], idx_g)
        pltpu.async_copy(tab_hbm.at[idx_g], rows_v, sem).wait()
        _cast_chunk(rows_v, out_v, G)
        pltpu.sync_copy(out_v, out_hbm.at[pl.ds((base + g * G) * D, G * D)])
        return ()

    lax.fori_loop(0, num_g, g_body, (), unroll=False)


@functools.partial(jax.jit, static_argnames=("interpret",))
def _run(idx_flat, embedding_weight, interpret=False):
    B = idx_flat.shape[0]
    rows_per_w = B // NW
    G = 128

    mesh = plsc.VectorSubcoreMesh(
        core_axis_name="c", subcore_axis_name="s", num_cores=NC, num_subcores=NS
    )
    kfn = pl.kernel(
        functools.partial(_body, rows_per_w, G),
        out_type=jax.ShapeDtypeStruct((B * D,), jnp.bfloat16),
        mesh=mesh,
        scratch_types=[
            pltpu.VMEM((G,), jnp.int32),
            pltpu.VMEM((G, D), jnp.float32),
            pltpu.VMEM((G * D,), jnp.bfloat16),
            pltpu.SemaphoreType.DMA,
        ],
        compiler_params=pltpu.CompilerParams(
            needs_layout_passes=False, use_tc_tiling_on_sc=False
        ),
        interpret=interpret,
    )
    return kfn(idx_flat, embedding_weight)


def kernel(input, embedding_weight, interpret=False):
    B, H = input.shape
    idx_flat = input.reshape(B * H)
    out = _run(idx_flat, embedding_weight, interpret=interpret)
    return out.reshape(B, H, D)


# speedup vs baseline: 1.4320x; 1.4320x over previous
import functools

import jax
import jax.numpy as jnp
from jax import lax
from jax.experimental import pallas as pl
from jax.experimental.pallas import tpu as pltpu
from jax.experimental.pallas import tpu_sc as plsc

NC, NS, L = 2, 16, 16
NW = NC * NS

D = 64
WPR = D // 2

V = 1_000_000
B = 4096
H = 200

RB = 2048
RB4 = RB // 4
NBLK = (V + RB - 1) // RB
VS = NBLK * RB


def _rne_bf16_bits(x):
    xi = jax.lax.bitcast_convert_type(x, jnp.uint32)
    return (xi + jnp.uint32(0x7FFF) + ((xi >> 16) & jnp.uint32(1))) >> 16


def _pack_tc_body(in_ref, out_ref):
    xlo = in_ref[0:WPR, :]
    xhi = in_ref[WPR:D, :]
    w = _rne_bf16_bits(xlo) | (_rne_bf16_bits(xhi) << 16)
    wq = jnp.concatenate(
        [w[:, p * RB4 : (p + 1) * RB4] for p in range(4)], axis=0
    )
    out_ref[...] = jax.lax.bitcast_convert_type(jnp.transpose(wq), jnp.int32)


def _pack_table(tabT):
    return pl.pallas_call(
        _pack_tc_body,
        grid=(NBLK,),
        in_specs=[pl.BlockSpec((D, RB), lambda i: (0, i))],
        out_specs=pl.BlockSpec((RB4, 128), lambda i: (i, 0)),
        out_shape=jax.ShapeDtypeStruct((NBLK * RB4, 128), jnp.int32),
    )(tabT)


GB = 128
NBUF = 4
BLOCKS = H * (B // GB)
BPW = BLOCKS // NW
GROUPS = BPW // NBUF


def _slot_map(idx_v, idx2_v):
    for g in range(GB // L):
        v = idx_v[pl.ds(g * L, L)]
        hi = v & jnp.int32(-2048)
        q4 = (v & jnp.int32(511)) << 2
        p = (v >> 9) & jnp.int32(3)
        idx2_v[pl.ds(g * L, L)] = hi | q4 | p


def _transpose_block(rows_v, tr_v):
    iota = lax.iota(jnp.int32, L)

    def row_body(r, _):
        rsp = jnp.full((L,), r, dtype=jnp.int32)
        for half in range(2):
            vals = rows_v[r, pl.ds(half * L, L)]
            plsc.store_scatter(tr_v, [iota + half * L, rsp], vals)
        return ()

    lax.fori_loop(0, GB, row_body, (), unroll=2)


def _gather_body(idxT_hbm, wtab_hbm, out_hbm, idx_v, idx2_v, rows_v, tr_v, sems):
    isems, gsems, osems = sems
    wid = lax.axis_index("s") * NC + lax.axis_index("c")
    base = wid * BPW

    def idx_copy(g, slot):
        h = g // (B // GB)
        j = g % (B // GB)
        return pltpu.make_async_copy(
            idxT_hbm.at[h, pl.ds(j * GB, GB)], idx_v.at[slot], isems[slot]
        )

    def gather_copy(slot):
        return pltpu.make_async_copy(
            wtab_hbm.at[idx2_v.at[slot]], rows_v.at[slot], gsems[slot]
        )

    def out_copy(g, slot):
        h = g // (B // GB)
        j = g % (B // GB)
        return pltpu.make_async_copy(
            tr_v.at[slot], out_hbm.at[h, :, pl.ds(j * GB, GB)], osems[slot]
        )

    for s in range(NBUF):
        idx_copy(base + s, s).start()

    def group_body(t, _):
        g0 = base + t * NBUF
        for s in range(NBUF):
            idx_copy(g0 + s, s).wait()
            _slot_map(idx_v.at[s], idx2_v.at[s])
            gather_copy(s).start()
        for s in range(NBUF):
            gather_copy(s).wait()

            @pl.when(t > 0)
            def _():
                out_copy(g0 - NBUF + s, s).wait()

            _transpose_block(rows_v.at[s], tr_v.at[s])
            out_copy(g0 + s, s).start()

            @pl.when(t < GROUPS - 1)
            def _():
                idx_copy(g0 + NBUF + s, s).start()

        return ()

    lax.fori_loop(0, GROUPS, group_body, (), unroll=False)
    for s in range(NBUF):
        out_copy(base + (GROUPS - 1) * NBUF + s, s).wait()


def _gather(idxT, wtab2):
    mesh = plsc.VectorSubcoreMesh(
        core_axis_name="c", subcore_axis_name="s", num_cores=NC, num_subcores=NS
    )
    kfn = pl.kernel(
        _gather_body,
        out_type=jax.ShapeDtypeStruct((H, WPR, B), jnp.int32),
        mesh=mesh,
        scratch_types=[
            pltpu.VMEM((NBUF, GB), jnp.int32),
            pltpu.VMEM((NBUF, GB), jnp.int32),
            pltpu.VMEM((NBUF, GB, WPR), jnp.int32),
            pltpu.VMEM((NBUF, WPR, GB), jnp.int32),
            (
                [pltpu.SemaphoreType.DMA] * NBUF,
                [pltpu.SemaphoreType.DMA] * NBUF,
                [pltpu.SemaphoreType.DMA] * NBUF,
            ),
        ],
        compiler_params=pltpu.CompilerParams(
            needs_layout_passes=False, use_tc_tiling_on_sc=False
        ),
    )
    return kfn(idxT, wtab2)


@jax.jit
def _run(input, embedding_weight):
    tabT = jnp.transpose(embedding_weight)
    wtab = _pack_table(tabT)
    wtab2 = wtab.reshape(VS, WPR)
    idxT = jnp.transpose(input)
    outw = _gather(idxT, wtab2)
    ob = jax.lax.bitcast_convert_type(outw, jnp.bfloat16)
    return jnp.transpose(ob, (2, 0, 3, 1)).reshape(B, H, D)


def kernel(input, embedding_weight):
    return _run(input, embedding_weight)


# speedup vs baseline: 1.5692x; 1.0958x over previous
import functools

import jax
import jax.numpy as jnp
from jax import lax
from jax.experimental import pallas as pl
from jax.experimental.pallas import tpu as pltpu
from jax.experimental.pallas import tpu_sc as plsc

NC, NS, L = 2, 16, 16
NW = NC * NS

D = 64
WPR = D // 2

V = 1_000_000
B = 4096
H = 200

RB = 2048
RB4 = RB // 4
NBLK = (V + RB - 1) // RB
VS = NBLK * RB


def _rne_bf16_bits(x):
    xi = jax.lax.bitcast_convert_type(x, jnp.uint32)
    return (xi + jnp.uint32(0x7FFF) + ((xi >> 16) & jnp.uint32(1))) >> 16


def _pack_tc_body(in_ref, out_ref):
    xlo = in_ref[0:WPR, :]
    xhi = in_ref[WPR:D, :]
    w = _rne_bf16_bits(xlo) | (_rne_bf16_bits(xhi) << 16)
    wq = jnp.concatenate(
        [w[:, p * RB4 : (p + 1) * RB4] for p in range(4)], axis=0
    )
    out_ref[...] = jax.lax.bitcast_convert_type(jnp.transpose(wq), jnp.int32)


def _pack_table(tabT):
    return pl.pallas_call(
        _pack_tc_body,
        grid=(NBLK,),
        in_specs=[pl.BlockSpec((D, RB), lambda i: (0, i))],
        out_specs=pl.BlockSpec((RB4, 128), lambda i: (i, 0)),
        out_shape=jax.ShapeDtypeStruct((NBLK * RB4, 128), jnp.int32),
    )(tabT)


GB = 128
NBUF = 8
BLOCKS = H * (B // GB)
BPW = BLOCKS // NW
GROUPS = BPW // NBUF
IPW = BPW * GB


def _slot_map_all(idx_v):
    def body(g, _):
        v = idx_v[pl.ds(g * L, L)]
        hi = v & jnp.int32(-2048)
        q4 = (v & jnp.int32(511)) << 2
        p = (v >> 9) & jnp.int32(3)
        idx_v[pl.ds(g * L, L)] = hi | q4 | p
        return ()

    lax.fori_loop(0, IPW // L, body, (), unroll=4)


def _cpair_block(rows_v, tr_v):
    iota2 = lax.iota(jnp.int32, L) * 2
    m16 = jnp.int32(0xFFFF)

    def row_body(r, _):
        rsp = jnp.full((L,), r, dtype=jnp.int32)
        we = plsc.load_gather(rows_v, [rsp, iota2])
        wo = plsc.load_gather(rows_v, [rsp, iota2 + 1])
        lo = (we & m16) | (wo << 16)
        hi = lax.shift_right_logical(we, 16) | (wo & ~m16)
        tr_v[r, pl.ds(0, 2 * L)] = plsc.bitcast(lo, jnp.bfloat16)
        tr_v[r, pl.ds(2 * L, 2 * L)] = plsc.bitcast(hi, jnp.bfloat16)
        return ()

    lax.fori_loop(0, GB, row_body, (), unroll=2)


def _gather_body(idx_hbm, wtab_hbm, out_hbm, idx_v, rows_v, tr_v, sems):
    isem, gsems, osems = sems
    wid = lax.axis_index("s") * NC + lax.axis_index("c")
    base = wid * BPW

    def gather_copy(t, slot):
        return pltpu.make_async_copy(
            wtab_hbm.at[idx_v.at[pl.ds((t - base) * GB, GB)]],
            rows_v.at[slot],
            gsems[slot],
        )

    def out_copy(t, slot):
        h = t // (B // GB)
        j = t % (B // GB)
        return pltpu.make_async_copy(
            tr_v.at[slot],
            out_hbm.at[pl.ds(j * GB, GB), h, :],
            osems[slot],
        )

    pltpu.make_async_copy(
        idx_hbm.at[pl.ds(wid * IPW, IPW)], idx_v, isem
    ).start()
    pltpu.make_async_copy(
        idx_hbm.at[pl.ds(wid * IPW, IPW)], idx_v, isem
    ).wait()
    _slot_map_all(idx_v)

    for s in range(NBUF):
        gather_copy(base + s, s).start()

    def group_body(g, _):
        t0 = base + g * NBUF
        for s in range(NBUF):
            t = t0 + s
            gather_copy(t, s).wait()

            @pl.when(g > 0)
            def _():
                out_copy(t - NBUF, s).wait()

            _cpair_block(rows_v.at[s], tr_v.at[s])
            out_copy(t, s).start()

            @pl.when(g < GROUPS - 1)
            def _():
                gather_copy(t + NBUF, s).start()

        return ()

    lax.fori_loop(0, GROUPS, group_body, (), unroll=False)
    for s in range(NBUF):
        out_copy(base + (GROUPS - 1) * NBUF + s, s).wait()


def _gather(idx_flat, wtab2):
    mesh = plsc.VectorSubcoreMesh(
        core_axis_name="c", subcore_axis_name="s", num_cores=NC, num_subcores=NS
    )
    kfn = pl.kernel(
        _gather_body,
        out_type=jax.ShapeDtypeStruct((B, H, D), jnp.bfloat16),
        mesh=mesh,
        scratch_types=[
            pltpu.VMEM((IPW,), jnp.int32),
            pltpu.VMEM((NBUF, GB, WPR), jnp.int32),
            pltpu.VMEM((NBUF, GB, D), jnp.bfloat16),
            (
                pltpu.SemaphoreType.DMA,
                [pltpu.SemaphoreType.DMA] * NBUF,
                [pltpu.SemaphoreType.DMA] * NBUF,
            ),
        ],
        compiler_params=pltpu.CompilerParams(
            needs_layout_passes=False, use_tc_tiling_on_sc=False
        ),
    )
    return kfn(idx_flat, wtab2)


@jax.jit
def _run(input, embedding_weight):
    tabT = jnp.transpose(embedding_weight)
    wtab = _pack_table(tabT)
    wtab2 = wtab.reshape(VS, WPR)
    idx_flat = jnp.transpose(input).reshape(B * H)
    return _gather(idx_flat, wtab2)


def kernel(input, embedding_weight):
    return _run(input, embedding_weight)
